# baseline (speedup 1.0000x reference)
import functools

import jax
import jax.numpy as jnp
from jax import lax
from jax.experimental import pallas as pl
from jax.experimental.pallas import tpu as pltpu
from jax.experimental.pallas import tpu_sc as plsc

GRID = 16
N_CENTERS = 256
DIM = 512
BATCH = 4096

TCA_SHARE = 3584
TCA_BLOCK = 896
TCB_SHARE = 0
TCB_BLOCK = 512
TC_SHARE = TCA_SHARE + TCB_SHARE
SC_SHARE = BATCH - TC_SHARE

NC = 2
NS = 16
L = 16
NW = NC * NS
B_PER_W = SC_SHARE // NW
N_ACC = 8


def _make_sc_kernel():
    mesh = plsc.VectorSubcoreMesh(core_axis_name="c", subcore_axis_name="s")

    @functools.partial(
        pl.kernel,
        mesh=mesh,
        out_type=jax.ShapeDtypeStruct((1, NW * L), jnp.float32),
        scratch_types=[
            pltpu.VMEM((B_PER_W,), jnp.int32),
            pltpu.VMEM((B_PER_W,), jnp.int32),
            pltpu.VMEM((B_PER_W,), jnp.int32),
            pltpu.VMEM((B_PER_W, DIM), jnp.float32),
            pltpu.VMEM((B_PER_W, DIM), jnp.float32),
            pltpu.VMEM((L,), jnp.float32),
            pltpu.SemaphoreType.DMA,
            pltpu.SemaphoreType.DMA,
            pltpu.SemaphoreType.DMA,
        ],
    )
    def sc_loss(x_hbm, coords_hbm, centers_hbm, out_hbm,
                lon_v, lat_v, idx_v, x_buf, c_buf, acc_v,
                sem_a, sem_g, sem_x):
        wid = lax.axis_index("s") * NC + lax.axis_index("c")
        base = TC_SHARE + wid * B_PER_W
        cp_lon = pltpu.async_copy(coords_hbm.at[0, pl.ds(base, B_PER_W)],
                                  lon_v, sem_a)
        cp_lat = pltpu.async_copy(coords_hbm.at[1, pl.ds(base, B_PER_W)],
                                  lat_v, sem_g)
        cp_x = pltpu.async_copy(x_hbm.at[pl.ds(base, B_PER_W)], x_buf, sem_x)
        cp_lon.wait()
        cp_lat.wait()
        for j in range(B_PER_W // L):
            s = pl.ds(j * L, L)
            idx_v[s] = lon_v[s] * GRID + lat_v[s]
        cp_g = pltpu.async_copy(centers_hbm.at[idx_v], c_buf, sem_g)

        def body_x(r, acc):
            acc = list(acc)
            for j in range(DIM // L):
                s = pl.ds(j * L, L)
                xv = x_buf[r, s]
                acc[j % N_ACC] = acc[j % N_ACC] + xv * xv
            return tuple(acc)

        def body_c(r, acc):
            acc = list(acc)
            for j in range(DIM // L):
                s = pl.ds(j * L, L)
                cv = c_buf[r, s]
                xv = x_buf[r, s]
                acc[j % N_ACC] = acc[j % N_ACC] + cv * (cv - (xv + xv))
            return tuple(acc)

        cp_x.wait()
        accs = lax.fori_loop(
            0, B_PER_W, body_x,
            tuple(jnp.zeros((L,), jnp.float32) for _ in range(N_ACC)))
        cp_g.wait()
        accs = lax.fori_loop(0, B_PER_W, body_c, accs)

        total = accs[0]
        for a in accs[1:]:
            total = total + a
        acc_v[...] = total
        pltpu.sync_copy(acc_v, out_hbm.at[0, pl.ds(wid * L, L)])

    return sc_loss


_sc_loss = _make_sc_kernel()


def _make_tc_body(block, row0):
    def _tc_body(x_ref, coords_ref, centers_ref, out_ref):
        i = pl.program_id(0)
        lon = coords_ref[0, pl.ds(row0 + i * block, block)]
        lat = coords_ref[1, pl.ds(row0 + i * block, block)]
        idx = lon * GRID + lat
        onehot = jnp.where(
            jax.lax.broadcasted_iota(jnp.int32, (block, N_CENTERS), 1)
            == idx[:, None],
            jnp.float32(1.0), jnp.float32(0.0))
        g = jnp.dot(onehot.astype(jnp.bfloat16),
                    centers_ref[...].astype(jnp.bfloat16),
                    preferred_element_type=jnp.float32)
        d = x_ref[...] - g
        part = jnp.sum(d * d, axis=0, keepdims=True)

        @pl.when(i == 0)
        def _():
            out_ref[...] = jnp.zeros_like(out_ref)

        out_ref[...] += part

    return _tc_body


def _make_tc_loss(share, block, row0):
    return pl.pallas_call(
        _make_tc_body(block, row0),
        grid=(share // block,),
        in_specs=[
            pl.BlockSpec((block, DIM), lambda i, r=row0 // block: (r + i, 0)),
            pl.BlockSpec((2, BATCH), lambda i: (0, 0)),
            pl.BlockSpec((N_CENTERS, DIM), lambda i: (0, 0)),
        ],
        out_specs=pl.BlockSpec((1, DIM), lambda i: (0, 0)),
        out_shape=jax.ShapeDtypeStruct((1, DIM), jnp.float32),
    )


_tc_loss_a = _make_tc_loss(TCA_SHARE, TCA_BLOCK, 0)


def kernel(batch_tensors, batch_coords, cluster_centers):
    tca_cols = _tc_loss_a(batch_tensors, batch_coords, cluster_centers)
    sc_partials = _sc_loss(batch_tensors, batch_coords, cluster_centers)
    return jnp.sum(tca_cols + sc_partials) / jnp.float32(BATCH * DIM)


# speedup vs baseline: 1.0154x; 1.0154x over previous
import functools

import jax
import jax.numpy as jnp
from jax import lax
from jax.experimental import pallas as pl
from jax.experimental.pallas import tpu as pltpu
from jax.experimental.pallas import tpu_sc as plsc

GRID = 16
N_CENTERS = 256
DIM = 512
BATCH = 4096

TCA_SHARE = 3584
TCA_BLOCK = 896
TCB_SHARE = 0
TCB_BLOCK = 512
TC_SHARE = TCA_SHARE + TCB_SHARE
SC_SHARE = BATCH - TC_SHARE

NC = 2
NS = 16
L = 16
NW = NC * NS
B_PER_W = SC_SHARE // NW
N_ACC = 8


def _make_sc_kernel():
    mesh = plsc.VectorSubcoreMesh(core_axis_name="c", subcore_axis_name="s")

    @functools.partial(
        pl.kernel,
        mesh=mesh,
        out_type=jax.ShapeDtypeStruct((1, NW * L), jnp.float32),
        scratch_types=[
            pltpu.VMEM((B_PER_W,), jnp.int32),
            pltpu.VMEM((B_PER_W,), jnp.int32),
            pltpu.VMEM((B_PER_W,), jnp.int32),
            pltpu.VMEM((B_PER_W, DIM), jnp.float32),
            pltpu.VMEM((B_PER_W, DIM), jnp.float32),
            pltpu.VMEM((L,), jnp.float32),
            pltpu.SemaphoreType.DMA,
            pltpu.SemaphoreType.DMA,
            pltpu.SemaphoreType.DMA,
        ],
    )
    def sc_loss(x_hbm, coords_hbm, centers_hbm, out_hbm,
                lon_v, lat_v, idx_v, x_buf, c_buf, acc_v,
                sem_a, sem_g, sem_x):
        wid = lax.axis_index("s") * NC + lax.axis_index("c")
        base = TC_SHARE + wid * B_PER_W
        cp_lon = pltpu.async_copy(coords_hbm.at[0, pl.ds(base, B_PER_W)],
                                  lon_v, sem_a)
        cp_lat = pltpu.async_copy(coords_hbm.at[1, pl.ds(base, B_PER_W)],
                                  lat_v, sem_g)
        cp_x = pltpu.async_copy(x_hbm.at[pl.ds(base, B_PER_W)], x_buf, sem_x)
        cp_lon.wait()
        cp_lat.wait()
        for j in range(B_PER_W // L):
            s = pl.ds(j * L, L)
            idx_v[s] = lon_v[s] * GRID + lat_v[s]
        cp_g = pltpu.async_copy(centers_hbm.at[idx_v], c_buf, sem_g)
        cp_g.wait()
        cp_x.wait()

        def body(r, acc):
            acc = list(acc)
            for j in range(DIM // L):
                s = pl.ds(j * L, L)
                d = x_buf[r, s] - c_buf[r, s]
                acc[j % N_ACC] = acc[j % N_ACC] + d * d
            return tuple(acc)

        accs = lax.fori_loop(
            0, B_PER_W, body,
            tuple(jnp.zeros((L,), jnp.float32) for _ in range(N_ACC)))

        total = accs[0]
        for a in accs[1:]:
            total = total + a
        acc_v[...] = total
        pltpu.sync_copy(acc_v, out_hbm.at[0, pl.ds(wid * L, L)])

    return sc_loss


_sc_loss = _make_sc_kernel()


def _make_tc_body(block, row0):
    def _tc_body(x_ref, coords_ref, centers_ref, out_ref):
        i = pl.program_id(0)
        lon = coords_ref[0, pl.ds(row0 + i * block, block)]
        lat = coords_ref[1, pl.ds(row0 + i * block, block)]
        idx = lon * GRID + lat
        onehot = jnp.where(
            jax.lax.broadcasted_iota(jnp.int32, (block, N_CENTERS), 1)
            == idx[:, None],
            jnp.float32(1.0), jnp.float32(0.0))
        g = jnp.dot(onehot.astype(jnp.bfloat16),
                    centers_ref[...].astype(jnp.bfloat16),
                    preferred_element_type=jnp.float32)
        d = x_ref[...] - g
        part = jnp.sum(d * d, axis=0, keepdims=True)

        @pl.when(i == 0)
        def _():
            out_ref[...] = jnp.zeros_like(out_ref)

        out_ref[...] += part

    return _tc_body


def _make_tc_loss(share, block, row0):
    return pl.pallas_call(
        _make_tc_body(block, row0),
        grid=(share // block,),
        in_specs=[
            pl.BlockSpec((block, DIM), lambda i, r=row0 // block: (r + i, 0)),
            pl.BlockSpec((2, BATCH), lambda i: (0, 0)),
            pl.BlockSpec((N_CENTERS, DIM), lambda i: (0, 0)),
        ],
        out_specs=pl.BlockSpec((1, DIM), lambda i: (0, 0)),
        out_shape=jax.ShapeDtypeStruct((1, DIM), jnp.float32),
    )


_tc_loss_a = _make_tc_loss(TCA_SHARE, TCA_BLOCK, 0)


def kernel(batch_tensors, batch_coords, cluster_centers):
    tca_cols = _tc_loss_a(batch_tensors, batch_coords, cluster_centers)
    sc_partials = _sc_loss(batch_tensors, batch_coords, cluster_centers)
    return jnp.sum(tca_cols + sc_partials) / jnp.float32(BATCH * DIM)


# speedup vs baseline: 1.0208x; 1.0053x over previous
import functools

import jax
import jax.numpy as jnp
from jax import lax
from jax.experimental import pallas as pl
from jax.experimental.pallas import tpu as pltpu
from jax.experimental.pallas import tpu_sc as plsc

GRID = 16
N_CENTERS = 256
DIM = 512
BATCH = 4096

TCA_SHARE = 3584
TCA_BLOCK = 896
TCB_SHARE = 0
TCB_BLOCK = 512
TC_SHARE = TCA_SHARE + TCB_SHARE
SC_SHARE = BATCH - TC_SHARE

NC = 2
NS = 16
L = 16
NW = NC * NS
B_PER_W = SC_SHARE // NW
N_ACC = 8


def _make_sc_kernel():
    mesh = plsc.VectorSubcoreMesh(core_axis_name="c", subcore_axis_name="s")

    @functools.partial(
        pl.kernel,
        mesh=mesh,
        out_type=jax.ShapeDtypeStruct((1, NW * L), jnp.float32),
        scratch_types=[
            pltpu.VMEM((B_PER_W,), jnp.int32),
            pltpu.VMEM((B_PER_W,), jnp.int32),
            pltpu.VMEM((B_PER_W,), jnp.int32),
            pltpu.VMEM((B_PER_W, DIM), jnp.float32),
            pltpu.VMEM((B_PER_W, DIM), jnp.float32),
            pltpu.VMEM((L,), jnp.float32),
            pltpu.SemaphoreType.DMA,
            pltpu.SemaphoreType.DMA,
            pltpu.SemaphoreType.DMA,
        ],
    )
    def sc_loss(x_hbm, coords_hbm, centers_hbm, out_hbm,
                lon_v, lat_v, idx_v, x_buf, c_buf, acc_v,
                sem_a, sem_g, sem_x):
        wid = lax.axis_index("s") * NC + lax.axis_index("c")
        base = TC_SHARE + wid * B_PER_W
        cp_lon = pltpu.async_copy(coords_hbm.at[0, pl.ds(base, B_PER_W)],
                                  lon_v, sem_a)
        cp_lat = pltpu.async_copy(coords_hbm.at[1, pl.ds(base, B_PER_W)],
                                  lat_v, sem_g)
        cp_x = pltpu.async_copy(x_hbm.at[pl.ds(base, B_PER_W)], x_buf, sem_x)
        cp_lon.wait()
        cp_lat.wait()
        for j in range(B_PER_W // L):
            s = pl.ds(j * L, L)
            idx_v[s] = lon_v[s] * GRID + lat_v[s]
        cp_g = pltpu.async_copy(centers_hbm.at[idx_v], c_buf, sem_g)
        cp_g.wait()
        cp_x.wait()

        def body(t, acc):
            r = t // (DIM // (L * N_ACC))
            g0 = (t % (DIM // (L * N_ACC))) * N_ACC
            acc = list(acc)
            for j in range(N_ACC):
                s = pl.ds((g0 + j) * L, L)
                d = x_buf[r, s] - c_buf[r, s]
                acc[j] = acc[j] + d * d
            return tuple(acc)

        accs = lax.fori_loop(
            0, B_PER_W * (DIM // (L * N_ACC)), body,
            tuple(jnp.zeros((L,), jnp.float32) for _ in range(N_ACC)))

        total = accs[0]
        for a in accs[1:]:
            total = total + a
        acc_v[...] = total
        pltpu.sync_copy(acc_v, out_hbm.at[0, pl.ds(wid * L, L)])

    return sc_loss


_sc_loss = _make_sc_kernel()


def _make_tc_body(block, row0):
    def _tc_body(x_ref, coords_ref, centers_ref, out_ref):
        i = pl.program_id(0)
        lon = coords_ref[0, pl.ds(row0 + i * block, block)]
        lat = coords_ref[1, pl.ds(row0 + i * block, block)]
        idx = lon * GRID + lat
        onehot = jnp.where(
            jax.lax.broadcasted_iota(jnp.int32, (block, N_CENTERS), 1)
            == idx[:, None],
            jnp.float32(1.0), jnp.float32(0.0))
        g = jnp.dot(onehot.astype(jnp.bfloat16),
                    centers_ref[...].astype(jnp.bfloat16),
                    preferred_element_type=jnp.float32)
        d = x_ref[...] - g
        part = jnp.sum(d * d, axis=0, keepdims=True)

        @pl.when(i == 0)
        def _():
            out_ref[...] = jnp.zeros_like(out_ref)

        out_ref[...] += part

    return _tc_body


def _make_tc_loss(share, block, row0):
    return pl.pallas_call(
        _make_tc_body(block, row0),
        grid=(share // block,),
        in_specs=[
            pl.BlockSpec((block, DIM), lambda i, r=row0 // block: (r + i, 0)),
            pl.BlockSpec((2, BATCH), lambda i: (0, 0)),
            pl.BlockSpec((N_CENTERS, DIM), lambda i: (0, 0)),
        ],
        out_specs=pl.BlockSpec((1, DIM), lambda i: (0, 0)),
        out_shape=jax.ShapeDtypeStruct((1, DIM), jnp.float32),
    )


_tc_loss_a = _make_tc_loss(TCA_SHARE, TCA_BLOCK, 0)


def kernel(batch_tensors, batch_coords, cluster_centers):
    tca_cols = _tc_loss_a(batch_tensors, batch_coords, cluster_centers)
    sc_partials = _sc_loss(batch_tensors, batch_coords, cluster_centers)
    return jnp.sum(tca_cols + sc_partials) / jnp.float32(BATCH * DIM)
